# Initial kernel scaffold; baseline (speedup 1.0000x reference)
import jax
import jax.numpy as jnp
from jax.experimental import pallas as pl


def kernel(x):
    raise NotImplementedError("write your pallas kernel here")


# speedup vs baseline: 1.2982x; 1.2982x over previous
import functools

import jax
import jax.numpy as jnp
from jax.experimental import pallas as pl
from jax.experimental.pallas import tpu as pltpu

_K = 24
_ITERS = 20
_TOL = 1e-06


def _kmeans_body(idx_ref, pts_ref, traj_ref, shift_ref):
    pts = pts_ref[0]
    n, c = pts.shape
    p2 = jnp.sum(pts * pts, axis=1, keepdims=True)
    pts_bf = pts.astype(jnp.bfloat16)

    idxv = idx_ref[0]
    iota_n = jax.lax.broadcasted_iota(jnp.int32, (_K, n), 1)
    oh0 = (iota_n == idxv).astype(jnp.float32)
    cent0 = jax.lax.dot_general(
        oh0, pts, (((1,), (0,)), ((), ())),
        precision=jax.lax.Precision.HIGHEST,
        preferred_element_type=jnp.float32)

    iota_k = jax.lax.broadcasted_iota(jnp.int32, (n, _K), 1)
    lane = jax.lax.broadcasted_iota(jnp.int32, (8, 128), 1)
    sub = jax.lax.broadcasted_iota(jnp.int32, (8, 128), 0)

    def body(i, carry):
        cent, svec = carry
        c2 = jnp.sum(cent * cent, axis=1)[None, :]
        dots = jax.lax.dot_general(
            pts_bf, cent.astype(jnp.bfloat16), (((1,), (1,)), ((), ())),
            preferred_element_type=jnp.float32)
        d = p2 + c2 - 2.0 * dots
        dmin = jnp.min(d, axis=1, keepdims=True)
        first = jnp.min(jnp.where(d == dmin, iota_k, _K), axis=1,
                        keepdims=True)
        oh = (iota_k == first).astype(jnp.float32)
        counts = jnp.sum(oh, axis=0)[:, None]
        sums = jax.lax.dot_general(
            oh.astype(jnp.bfloat16), pts_bf, (((0,), (0,)), ((), ())),
            preferred_element_type=jnp.float32)
        new = jnp.where(counts > 0.0,
                        sums / jnp.maximum(counts, 1.0), cent)
        diff = new - cent
        shift = jnp.sum(diff * diff) * (1.0 / _K)
        svec = jnp.where((lane == i) & (sub == 0), shift, svec)
        traj_ref[0, i] = new
        return new, svec

    _, svec = jax.lax.fori_loop(
        0, _ITERS, body, (cent0, jnp.zeros((8, 128), jnp.float32)))
    shift_ref[0] = svec


@functools.partial(jax.jit, static_argnames=())
def kernel(x):
    b, h, w, c = x.shape
    n = h * w
    pts = x.reshape(b, n, c)

    keys = jax.random.split(jax.random.key(42), b)
    idx = jnp.stack([jax.random.permutation(k, n)[:_K] for k in keys])
    idx = idx.astype(jnp.int32).reshape(b, _K, 1)

    traj, shifts = pl.pallas_call(
        _kmeans_body,
        grid=(b,),
        in_specs=[
            pl.BlockSpec((1, _K, 1), lambda i: (i, 0, 0)),
            pl.BlockSpec((1, n, c), lambda i: (i, 0, 0)),
        ],
        out_specs=[
            pl.BlockSpec((1, _ITERS, _K, c), lambda i: (i, 0, 0, 0)),
            pl.BlockSpec((1, 8, 128), lambda i: (i, 0, 0)),
        ],
        out_shape=[
            jax.ShapeDtypeStruct((b, _ITERS, _K, c), jnp.float32),
            jax.ShapeDtypeStruct((b, 8, 128), jnp.float32),
        ],
    )(idx, pts)

    s = jnp.mean(shifts[:, 0, :_ITERS], axis=0)
    le = s <= _TOL
    t = jnp.where(jnp.any(le), jnp.argmax(le), _ITERS - 1)
    return jax.lax.dynamic_index_in_dim(traj, t, axis=1, keepdims=False)


# speedup vs baseline: 1.9357x; 1.4911x over previous
import functools

import jax
import jax.numpy as jnp
from jax.experimental import pallas as pl
from jax.experimental.pallas import tpu as pltpu

_K = 24
_ITERS = 20
_TOL = 1e-06


def _kmeans_body(idx_ref, pts_ref, ptsT_ref, traj_ref, shift_ref):
    pts = pts_ref[0]
    ptsT = ptsT_ref[0]
    n, c = pts.shape
    p2r = jnp.sum(ptsT * ptsT, axis=0, keepdims=True)
    pts_bf = pts.astype(jnp.bfloat16)
    ptsT_bf = ptsT.astype(jnp.bfloat16)

    idxv = idx_ref[0]
    iota_n = jax.lax.broadcasted_iota(jnp.int32, (_K, n), 1)
    oh0 = (iota_n == idxv).astype(jnp.float32)
    cent0 = jax.lax.dot_general(
        oh0, pts, (((1,), (0,)), ((), ())),
        precision=jax.lax.Precision.HIGHEST,
        preferred_element_type=jnp.float32)

    iota_k = jax.lax.broadcasted_iota(jnp.int32, (_K, n), 0)
    ones_col = jnp.ones((n, 1), jnp.bfloat16)
    lane = jax.lax.broadcasted_iota(jnp.int32, (8, 128), 1)
    sub = jax.lax.broadcasted_iota(jnp.int32, (8, 128), 0)

    def body(i, carry):
        cent, svec = carry
        c2 = jnp.sum(cent * cent, axis=1)[:, None]
        dots = jax.lax.dot_general(
            cent.astype(jnp.bfloat16), ptsT_bf, (((1,), (0,)), ((), ())),
            preferred_element_type=jnp.float32)
        d = p2r + c2 - 2.0 * dots
        dmin = jnp.min(d, axis=0, keepdims=True)
        first = jnp.min(jnp.where(d == dmin, iota_k, _K), axis=0,
                        keepdims=True)
        oh = (iota_k == first).astype(jnp.bfloat16)
        counts = jax.lax.dot_general(
            oh, ones_col, (((1,), (0,)), ((), ())),
            preferred_element_type=jnp.float32)
        sums = jax.lax.dot_general(
            oh, pts_bf, (((1,), (0,)), ((), ())),
            preferred_element_type=jnp.float32)
        new = jnp.where(counts > 0.0,
                        sums / jnp.maximum(counts, 1.0), cent)
        diff = new - cent
        shift = jnp.sum(diff * diff) * (1.0 / _K)
        svec = jnp.where((lane == i) & (sub == 0), shift, svec)
        traj_ref[0, i] = new
        return new, svec

    _, svec = jax.lax.fori_loop(
        0, _ITERS, body, (cent0, jnp.zeros((8, 128), jnp.float32)))
    shift_ref[0] = svec


@functools.partial(jax.jit, static_argnames=())
def kernel(x):
    b, h, w, c = x.shape
    n = h * w
    pts = x.reshape(b, n, c)

    keys = jax.random.split(jax.random.key(42), b)
    idx = jnp.stack([jax.random.permutation(k, n)[:_K] for k in keys])
    idx = idx.astype(jnp.int32).reshape(b, _K, 1)

    ptsT = jnp.swapaxes(pts, 1, 2)

    traj, shifts = pl.pallas_call(
        _kmeans_body,
        grid=(b,),
        in_specs=[
            pl.BlockSpec((1, _K, 1), lambda i: (i, 0, 0)),
            pl.BlockSpec((1, n, c), lambda i: (i, 0, 0)),
            pl.BlockSpec((1, c, n), lambda i: (i, 0, 0)),
        ],
        out_specs=[
            pl.BlockSpec((1, _ITERS, _K, c), lambda i: (i, 0, 0, 0)),
            pl.BlockSpec((1, 8, 128), lambda i: (i, 0, 0)),
        ],
        out_shape=[
            jax.ShapeDtypeStruct((b, _ITERS, _K, c), jnp.float32),
            jax.ShapeDtypeStruct((b, 8, 128), jnp.float32),
        ],
    )(idx, pts, ptsT)

    s = jnp.mean(shifts[:, 0, :_ITERS], axis=0)
    le = s <= _TOL
    t = jnp.where(jnp.any(le), jnp.argmax(le), _ITERS - 1)
    return jax.lax.dynamic_index_in_dim(traj, t, axis=1, keepdims=False)


# speedup vs baseline: 2.0775x; 1.0732x over previous
import functools

import jax
import jax.numpy as jnp
from jax.experimental import pallas as pl
from jax.experimental.pallas import tpu as pltpu

_K = 24
_ITERS = 20
_TOL = 1e-06


def _kmeans_body(idx_ref, pts_ref, ptsT_ref, traj_ref, shift_ref):
    pts = pts_ref[0]
    ptsT = ptsT_ref[0]
    n, c = pts.shape
    p2r = jnp.sum(ptsT * ptsT, axis=0, keepdims=True)
    pts_bf = pts.astype(jnp.bfloat16)
    ptsT_bf = ptsT.astype(jnp.bfloat16)

    idxv = idx_ref[0]
    iota_n = jax.lax.broadcasted_iota(jnp.int32, (_K, n), 1)
    oh0 = (iota_n == idxv).astype(jnp.float32)
    cent0 = jax.lax.dot_general(
        oh0, pts, (((1,), (0,)), ((), ())),
        precision=jax.lax.Precision.HIGHEST,
        preferred_element_type=jnp.float32)

    iota_k = jax.lax.broadcasted_iota(jnp.int32, (_K, n), 0)
    lane = jax.lax.broadcasted_iota(jnp.int32, (8, 128), 1)
    sub = jax.lax.broadcasted_iota(jnp.int32, (8, 128), 0)

    def body(i, carry):
        cent, svec = carry
        c2 = jnp.sum(cent * cent, axis=1)[:, None]
        dots = jax.lax.dot_general(
            cent.astype(jnp.bfloat16), ptsT_bf, (((1,), (0,)), ((), ())),
            preferred_element_type=jnp.float32)
        d = p2r + c2 - 2.0 * dots
        dmin = jnp.min(d, axis=0, keepdims=True)
        first = jnp.min(jnp.where(d == dmin, iota_k, _K), axis=0,
                        keepdims=True)
        ohm = iota_k == first
        oh = ohm.astype(jnp.bfloat16)
        counts = jnp.sum(ohm.astype(jnp.float32), axis=1,
                         keepdims=True)
        sums = jax.lax.dot_general(
            oh, pts_bf, (((1,), (0,)), ((), ())),
            preferred_element_type=jnp.float32)
        new = jnp.where(counts > 0.0,
                        sums / jnp.maximum(counts, 1.0), cent)
        diff = new - cent
        shift = jnp.sum(diff * diff) * (1.0 / _K)
        svec = jnp.where((lane == i) & (sub == 0), shift, svec)
        traj_ref[0, i] = new
        return new, svec

    _, svec = jax.lax.fori_loop(
        0, _ITERS, body, (cent0, jnp.zeros((8, 128), jnp.float32)))
    shift_ref[0] = svec


@functools.partial(jax.jit, static_argnames=())
def kernel(x):
    b, h, w, c = x.shape
    n = h * w
    pts = x.reshape(b, n, c)

    keys = jax.random.split(jax.random.key(42), b)
    idx = jnp.stack([jax.random.permutation(k, n)[:_K] for k in keys])
    idx = idx.astype(jnp.int32).reshape(b, _K, 1)

    ptsT = jnp.swapaxes(pts, 1, 2)

    traj, shifts = pl.pallas_call(
        _kmeans_body,
        grid=(b,),
        in_specs=[
            pl.BlockSpec((1, _K, 1), lambda i: (i, 0, 0)),
            pl.BlockSpec((1, n, c), lambda i: (i, 0, 0)),
            pl.BlockSpec((1, c, n), lambda i: (i, 0, 0)),
        ],
        out_specs=[
            pl.BlockSpec((1, _ITERS, _K, c), lambda i: (i, 0, 0, 0)),
            pl.BlockSpec((1, 8, 128), lambda i: (i, 0, 0)),
        ],
        out_shape=[
            jax.ShapeDtypeStruct((b, _ITERS, _K, c), jnp.float32),
            jax.ShapeDtypeStruct((b, 8, 128), jnp.float32),
        ],
        compiler_params=pltpu.CompilerParams(
            dimension_semantics=("parallel",)),
    )(idx, pts, ptsT)

    s = jnp.mean(shifts[:, 0, :_ITERS], axis=0)
    le = s <= _TOL
    t = jnp.where(jnp.any(le), jnp.argmax(le), _ITERS - 1)
    return jax.lax.dynamic_index_in_dim(traj, t, axis=1, keepdims=False)


# speedup vs baseline: 2.2309x; 1.0738x over previous
import functools

import jax
import jax.numpy as jnp
from jax.experimental import pallas as pl
from jax.experimental.pallas import tpu as pltpu

_K = 24
_ITERS = 20
_TOL = 1e-06


def _kmeans_body(idx_ref, pts_ref, traj_ref, shift_ref, ptsT_scr):
    pts = pts_ref[0]
    n, c = pts.shape
    chunk = 512
    for j in range(n // chunk):
        ptsT_scr[:, j * chunk:(j + 1) * chunk] = jnp.swapaxes(
            pts[j * chunk:(j + 1) * chunk, :], 0, 1)
    ptsT = ptsT_scr[...]
    p2r = jnp.sum(ptsT * ptsT, axis=0, keepdims=True)
    pts_bf = pts.astype(jnp.bfloat16)
    ptsT_bf = ptsT.astype(jnp.bfloat16)

    idxv = idx_ref[0]
    iota_n = jax.lax.broadcasted_iota(jnp.int32, (_K, n), 1)
    oh0 = (iota_n == idxv).astype(jnp.float32)
    cent0 = jax.lax.dot_general(
        oh0, pts, (((1,), (0,)), ((), ())),
        precision=jax.lax.Precision.HIGHEST,
        preferred_element_type=jnp.float32)

    iota_k = jax.lax.broadcasted_iota(jnp.int32, (_K, n), 0)
    lane = jax.lax.broadcasted_iota(jnp.int32, (8, 128), 1)
    sub = jax.lax.broadcasted_iota(jnp.int32, (8, 128), 0)

    def body(i, carry):
        cent, svec = carry
        c2 = jnp.sum(cent * cent, axis=1)[:, None]
        dots = jax.lax.dot_general(
            cent.astype(jnp.bfloat16), ptsT_bf, (((1,), (0,)), ((), ())),
            preferred_element_type=jnp.float32)
        d = p2r + c2 - 2.0 * dots
        dmin = jnp.min(d, axis=0, keepdims=True)
        first = jnp.min(jnp.where(d == dmin, iota_k, _K), axis=0,
                        keepdims=True)
        ohm = iota_k == first
        oh = ohm.astype(jnp.bfloat16)
        counts = jnp.sum(ohm.astype(jnp.float32), axis=1,
                         keepdims=True)
        sums = jax.lax.dot_general(
            oh, pts_bf, (((1,), (0,)), ((), ())),
            preferred_element_type=jnp.float32)
        new = jnp.where(counts > 0.0,
                        sums / jnp.maximum(counts, 1.0), cent)
        diff = new - cent
        shift = jnp.sum(diff * diff) * (1.0 / _K)
        svec = jnp.where((lane == i) & (sub == 0), shift, svec)
        traj_ref[0, i] = new
        return new, svec

    _, svec = jax.lax.fori_loop(
        0, _ITERS, body, (cent0, jnp.zeros((8, 128), jnp.float32)))
    shift_ref[0] = svec


@functools.partial(jax.jit, static_argnames=())
def kernel(x):
    b, h, w, c = x.shape
    n = h * w
    pts = x.reshape(b, n, c)

    keys = jax.random.split(jax.random.key(42), b)
    idx = jnp.stack([jax.random.permutation(k, n)[:_K] for k in keys])
    idx = idx.astype(jnp.int32).reshape(b, _K, 1)

    traj, shifts = pl.pallas_call(
        _kmeans_body,
        grid=(b,),
        in_specs=[
            pl.BlockSpec((1, _K, 1), lambda i: (i, 0, 0)),
            pl.BlockSpec((1, n, c), lambda i: (i, 0, 0)),
        ],
        scratch_shapes=[pltpu.VMEM((c, n), jnp.float32)],
        out_specs=[
            pl.BlockSpec((1, _ITERS, _K, c), lambda i: (i, 0, 0, 0)),
            pl.BlockSpec((1, 8, 128), lambda i: (i, 0, 0)),
        ],
        out_shape=[
            jax.ShapeDtypeStruct((b, _ITERS, _K, c), jnp.float32),
            jax.ShapeDtypeStruct((b, 8, 128), jnp.float32),
        ],
        compiler_params=pltpu.CompilerParams(
            dimension_semantics=("parallel",)),
    )(idx, pts)

    s = jnp.mean(shifts[:, 0, :_ITERS], axis=0)
    le = s <= _TOL
    t = jnp.where(jnp.any(le), jnp.argmax(le), _ITERS - 1)
    return jax.lax.dynamic_index_in_dim(traj, t, axis=1, keepdims=False)


# speedup vs baseline: 2.2825x; 1.0231x over previous
import functools

import jax
import jax.numpy as jnp
from jax.experimental import pallas as pl
from jax.experimental.pallas import tpu as pltpu

_K = 24
_ITERS = 20
_TOL = 1e-06


def _kmeans_body(idx_ref, pts_ref, traj_ref, shift_ref, ptsT_scr):
    pts = pts_ref[0]
    n, c = pts.shape
    b = pl.program_id(0)
    chunk = 512
    for j in range(n // chunk):
        ptsT_scr[:, j * chunk:(j + 1) * chunk] = jnp.swapaxes(
            pts[j * chunk:(j + 1) * chunk, :], 0, 1)
    ptsT = ptsT_scr[...]
    p2r = jnp.sum(ptsT * ptsT, axis=0, keepdims=True)
    pts_bf = pts.astype(jnp.bfloat16)
    ptsT_bf = ptsT.astype(jnp.bfloat16)

    cent0 = jnp.concatenate(
        [pts_ref[0, pl.ds(idx_ref[b, k], 1), :] for k in range(_K)],
        axis=0)

    iota_k = jax.lax.broadcasted_iota(jnp.int32, (_K, n), 0)
    lane = jax.lax.broadcasted_iota(jnp.int32, (8, 128), 1)
    sub = jax.lax.broadcasted_iota(jnp.int32, (8, 128), 0)

    def body(i, carry):
        cent, svec = carry
        c2 = jnp.sum(cent * cent, axis=1)[:, None]
        dots = jax.lax.dot_general(
            cent.astype(jnp.bfloat16), ptsT_bf, (((1,), (0,)), ((), ())),
            preferred_element_type=jnp.float32)
        d = p2r + c2 - 2.0 * dots
        dmin = jnp.min(d, axis=0, keepdims=True)
        first = jnp.min(jnp.where(d == dmin, iota_k, _K), axis=0,
                        keepdims=True)
        ohm = iota_k == first
        oh = ohm.astype(jnp.bfloat16)
        counts = jnp.sum(ohm.astype(jnp.float32), axis=1,
                         keepdims=True)
        sums = jax.lax.dot_general(
            oh, pts_bf, (((1,), (0,)), ((), ())),
            preferred_element_type=jnp.float32)
        new = jnp.where(counts > 0.0,
                        sums / jnp.maximum(counts, 1.0), cent)
        diff = new - cent
        shift = jnp.sum(diff * diff) * (1.0 / _K)
        svec = jnp.where((lane == i) & (sub == 0), shift, svec)
        traj_ref[0, i] = new
        return new, svec

    _, svec = jax.lax.fori_loop(
        0, _ITERS, body, (cent0, jnp.zeros((8, 128), jnp.float32)))
    shift_ref[0] = svec


@functools.partial(jax.jit, static_argnames=())
def kernel(x):
    b, h, w, c = x.shape
    n = h * w
    pts = x.reshape(b, n, c)

    keys = jax.random.split(jax.random.key(42), b)
    idx = jnp.stack([jax.random.permutation(k, n)[:_K] for k in keys])
    idx = idx.astype(jnp.int32)

    traj, shifts = pl.pallas_call(
        _kmeans_body,
        grid_spec=pltpu.PrefetchScalarGridSpec(
            num_scalar_prefetch=1,
            grid=(b,),
            in_specs=[
                pl.BlockSpec((1, n, c), lambda i, idx_ref: (i, 0, 0)),
            ],
            out_specs=[
                pl.BlockSpec((1, _ITERS, _K, c),
                             lambda i, idx_ref: (i, 0, 0, 0)),
                pl.BlockSpec((1, 8, 128), lambda i, idx_ref: (i, 0, 0)),
            ],
            scratch_shapes=[pltpu.VMEM((c, n), jnp.float32)],
        ),
        out_shape=[
            jax.ShapeDtypeStruct((b, _ITERS, _K, c), jnp.float32),
            jax.ShapeDtypeStruct((b, 8, 128), jnp.float32),
        ],
        compiler_params=pltpu.CompilerParams(
            dimension_semantics=("parallel",)),
    )(idx, pts)

    s = jnp.mean(shifts[:, 0, :_ITERS], axis=0)
    le = s <= _TOL
    t = jnp.where(jnp.any(le), jnp.argmax(le), _ITERS - 1)
    return jax.lax.dynamic_index_in_dim(traj, t, axis=1, keepdims=False)


# speedup vs baseline: 4.8487x; 2.1243x over previous
import functools

import jax
import jax.numpy as jnp
from jax.experimental import pallas as pl
from jax.experimental.pallas import tpu as pltpu

_K = 24
_ITERS = 20
_TOL = 1e-06


def _kmeans_body(idx_ref, pts_ref, traj_ref, shift_ref, ptsT_scr):
    pts = pts_ref[0]
    n, c = pts.shape
    b = pl.program_id(0)
    chunk = 512
    for j in range(n // chunk):
        ptsT_scr[:, j * chunk:(j + 1) * chunk] = jnp.swapaxes(
            pts[j * chunk:(j + 1) * chunk, :], 0, 1)
    ptsT = ptsT_scr[...]
    p2r = jnp.sum(ptsT * ptsT, axis=0, keepdims=True)
    pts_bf = pts.astype(jnp.bfloat16)
    ptsT_bf = ptsT.astype(jnp.bfloat16)

    cent0 = jnp.concatenate(
        [pts_ref[0, pl.ds(idx_ref[b, k], 1), :] for k in range(_K)],
        axis=0)

    iota_k = jax.lax.broadcasted_iota(jnp.int32, (_K, n), 0)
    lane = jax.lax.broadcasted_iota(jnp.int32, (8, 128), 1)
    sub = jax.lax.broadcasted_iota(jnp.int32, (8, 128), 0)

    def body(i, carry):
        cent, svec = carry
        c2 = jnp.sum(cent * cent, axis=1)[:, None]
        dots = jax.lax.dot_general(
            cent.astype(jnp.bfloat16), ptsT_bf, (((1,), (0,)), ((), ())),
            preferred_element_type=jnp.float32)
        d = p2r + c2 - 2.0 * dots
        dmin = jnp.min(d, axis=0, keepdims=True)
        first = jnp.min(jnp.where(d == dmin, iota_k, _K), axis=0,
                        keepdims=True)
        ohm = iota_k == first
        oh = ohm.astype(jnp.bfloat16)
        counts = jnp.sum(ohm.astype(jnp.float32), axis=1,
                         keepdims=True)
        sums = jax.lax.dot_general(
            oh, pts_bf, (((1,), (0,)), ((), ())),
            preferred_element_type=jnp.float32)
        new = jnp.where(counts > 0.0,
                        sums / jnp.maximum(counts, 1.0), cent)
        diff = new - cent
        shift = jnp.sum(diff * diff) * (1.0 / _K)
        svec = jnp.where((lane == i) & (sub == 0), shift, svec)
        traj_ref[0, i] = new
        return new, svec

    _, svec = jax.lax.fori_loop(
        0, _ITERS, body, (cent0, jnp.zeros((8, 128), jnp.float32)))
    shift_ref[0] = svec


@functools.partial(jax.jit, static_argnames=())
def kernel(x):
    b, h, w, c = x.shape
    n = h * w
    pts = x.reshape(b, n, c)

    with jax.ensure_compile_time_eval():
        keys = jax.random.split(jax.random.key(42), b)
        idx = jnp.stack([jax.random.permutation(k, n)[:_K] for k in keys])
        idx = idx.astype(jnp.int32)

    traj, shifts = pl.pallas_call(
        _kmeans_body,
        grid_spec=pltpu.PrefetchScalarGridSpec(
            num_scalar_prefetch=1,
            grid=(b,),
            in_specs=[
                pl.BlockSpec((1, n, c), lambda i, idx_ref: (i, 0, 0)),
            ],
            out_specs=[
                pl.BlockSpec((1, _ITERS, _K, c),
                             lambda i, idx_ref: (i, 0, 0, 0)),
                pl.BlockSpec((1, 8, 128), lambda i, idx_ref: (i, 0, 0)),
            ],
            scratch_shapes=[pltpu.VMEM((c, n), jnp.float32)],
        ),
        out_shape=[
            jax.ShapeDtypeStruct((b, _ITERS, _K, c), jnp.float32),
            jax.ShapeDtypeStruct((b, 8, 128), jnp.float32),
        ],
        compiler_params=pltpu.CompilerParams(
            dimension_semantics=("parallel",)),
    )(idx, pts)

    s = jnp.mean(shifts[:, 0, :_ITERS], axis=0)
    le = s <= _TOL
    t = jnp.where(jnp.any(le), jnp.argmax(le), _ITERS - 1)
    return jax.lax.dynamic_index_in_dim(traj, t, axis=1, keepdims=False)


# speedup vs baseline: 5.3891x; 1.1114x over previous
import functools

import jax
import jax.numpy as jnp
from jax.experimental import pallas as pl
from jax.experimental.pallas import tpu as pltpu

_K = 24
_ITERS = 20
_TOL = 1e-06
_PER = 2


def _kmeans_body(idx_ref, pts_ref, traj_ref, shift_ref, ptsT_scr, ptsb_scr):
    _, n, c = pts_ref.shape
    g = pl.program_id(0)
    chunk = 512

    def prep(s):
        parts = []
        for j in range(n // chunk):
            blk = pts_ref[s, pl.ds(j * chunk, chunk), :]
            tj = jnp.swapaxes(blk, 0, 1)
            parts.append(jnp.sum(tj * tj, axis=0, keepdims=True))
            ptsT_scr[s, :, j * chunk:(j + 1) * chunk] = tj.astype(jnp.bfloat16)
            ptsb_scr[s, j * chunk:(j + 1) * chunk, :] = blk.astype(jnp.bfloat16)
        p2r = jnp.concatenate(parts, axis=1)
        bb = g * _PER + s
        cent0 = jnp.concatenate(
            [pts_ref[s, pl.ds(idx_ref[bb, k], 1), :] for k in range(_K)],
            axis=0)
        return p2r, cent0

    states = [prep(s) for s in range(_PER)]

    iota_k = jax.lax.broadcasted_iota(jnp.int32, (_K, n), 0)
    lane = jax.lax.broadcasted_iota(jnp.int32, (8, 128), 1)
    sub = jax.lax.broadcasted_iota(jnp.int32, (8, 128), 0)

    def step(i, s, cent, svec, p2r):
        c2 = jnp.sum(cent * cent, axis=1)[:, None]
        dots = jax.lax.dot_general(
            cent.astype(jnp.bfloat16), ptsT_scr[s],
            (((1,), (0,)), ((), ())),
            preferred_element_type=jnp.float32)
        d = p2r + c2 - 2.0 * dots
        dmin = jnp.min(d, axis=0, keepdims=True)
        first = jnp.min(jnp.where(d == dmin, iota_k, _K), axis=0,
                        keepdims=True)
        ohm = iota_k == first
        counts = jnp.sum(ohm.astype(jnp.float32), axis=1,
                         keepdims=True)
        sums = jax.lax.dot_general(
            ohm.astype(jnp.bfloat16), ptsb_scr[s],
            (((1,), (0,)), ((), ())),
            preferred_element_type=jnp.float32)
        new = jnp.where(counts > 0.0,
                        sums / jnp.maximum(counts, 1.0), cent)
        diff = new - cent
        shift = jnp.sum(diff * diff) * (1.0 / _K)
        svec = jnp.where((lane == i) & (sub == 0), shift, svec)
        traj_ref[s, i] = new
        return new, svec

    def body(i, carry):
        out = []
        for s in range(_PER):
            cent, svec = carry[2 * s], carry[2 * s + 1]
            new, svec = step(i, s, cent, svec, states[s][0])
            out += [new, svec]
        return tuple(out)

    init = []
    for s in range(_PER):
        init += [states[s][1], jnp.zeros((8, 128), jnp.float32)]
    fin = jax.lax.fori_loop(0, _ITERS, body, tuple(init))
    for s in range(_PER):
        shift_ref[s] = fin[2 * s + 1]


def _init_indices(b, n):
    def build():
        keys = jax.random.split(jax.random.key(42), b)
        idx = jnp.stack([jax.random.permutation(k, n)[:_K] for k in keys])
        return idx.astype(jnp.int32)
    try:
        with jax.ensure_compile_time_eval():
            return build()
    except Exception:
        return build()


@functools.partial(jax.jit, static_argnames=())
def kernel(x):
    b, h, w, c = x.shape
    n = h * w
    pts = x.reshape(b, n, c)
    idx = _init_indices(b, n)

    traj, shifts = pl.pallas_call(
        _kmeans_body,
        grid_spec=pltpu.PrefetchScalarGridSpec(
            num_scalar_prefetch=1,
            grid=(b // _PER,),
            in_specs=[
                pl.BlockSpec((_PER, n, c), lambda i, idx_ref: (i, 0, 0)),
            ],
            out_specs=[
                pl.BlockSpec((_PER, _ITERS, _K, c),
                             lambda i, idx_ref: (i, 0, 0, 0)),
                pl.BlockSpec((_PER, 8, 128), lambda i, idx_ref: (i, 0, 0)),
            ],
            scratch_shapes=[
                pltpu.VMEM((_PER, c, n), jnp.bfloat16),
                pltpu.VMEM((_PER, n, c), jnp.bfloat16),
            ],
        ),
        out_shape=[
            jax.ShapeDtypeStruct((b, _ITERS, _K, c), jnp.float32),
            jax.ShapeDtypeStruct((b, 8, 128), jnp.float32),
        ],
        compiler_params=pltpu.CompilerParams(
            dimension_semantics=("parallel",)),
    )(idx, pts)

    s = jnp.mean(shifts[:, 0, :_ITERS], axis=0)
    le = s <= _TOL
    t = jnp.where(jnp.any(le), jnp.argmax(le), _ITERS - 1)
    return jax.lax.dynamic_index_in_dim(traj, t, axis=1, keepdims=False)


# speedup vs baseline: 5.3904x; 1.0002x over previous
import functools

import jax
import jax.numpy as jnp
from jax.experimental import pallas as pl
from jax.experimental.pallas import tpu as pltpu

_K = 24
_ITERS = 20
_TOL = 1e-06
_PER = 2


def _kmeans_body(idx_ref, pts_ref, traj_ref, shift_ref, ptsT_scr, ptsb_scr):
    _, n, c = pts_ref.shape
    g = pl.program_id(0)
    chunk = 512

    def prep(s):
        parts = []
        for j in range(n // chunk):
            blk = pts_ref[s, pl.ds(j * chunk, chunk), :]
            tj = jnp.swapaxes(blk, 0, 1)
            parts.append(jnp.sum(tj * tj, axis=0, keepdims=True))
            ptsT_scr[s, :, j * chunk:(j + 1) * chunk] = tj.astype(jnp.bfloat16)
            ptsb_scr[s, j * chunk:(j + 1) * chunk, :] = blk.astype(jnp.bfloat16)
        p2r = jnp.concatenate(parts, axis=1)
        bb = g * _PER + s
        cent0 = jnp.concatenate(
            [pts_ref[s, pl.ds(idx_ref[bb, k], 1), :] for k in range(_K)],
            axis=0)
        return p2r, cent0

    states = [prep(s) for s in range(_PER)]

    lane = jax.lax.broadcasted_iota(jnp.int32, (8, 128), 1)
    sub = jax.lax.broadcasted_iota(jnp.int32, (8, 128), 0)

    nq = 4
    q = n // nq
    iota_q = jax.lax.broadcasted_iota(jnp.int32, (_K, q), 0)

    def step(i, s, cent, svec, p2r):
        c2 = jnp.sum(cent * cent, axis=1)[:, None]
        cent_bf = cent.astype(jnp.bfloat16)
        oh_parts = []
        for h in range(nq):
            dots = jax.lax.dot_general(
                cent_bf, ptsT_scr[s, :, h * q:(h + 1) * q],
                (((1,), (0,)), ((), ())),
                preferred_element_type=jnp.float32)
            d = p2r[:, h * q:(h + 1) * q] + c2 - 2.0 * dots
            dmin = jnp.min(d, axis=0, keepdims=True)
            first = jnp.min(jnp.where(d == dmin, iota_q, _K), axis=0,
                            keepdims=True)
            oh_parts.append(iota_q == first)
        ohm = jnp.concatenate(oh_parts, axis=1)
        counts = jnp.sum(ohm.astype(jnp.float32), axis=1,
                         keepdims=True)
        sums = jax.lax.dot_general(
            ohm.astype(jnp.bfloat16), ptsb_scr[s],
            (((1,), (0,)), ((), ())),
            preferred_element_type=jnp.float32)
        new = jnp.where(counts > 0.0,
                        sums / jnp.maximum(counts, 1.0), cent)
        diff = new - cent
        shift = jnp.sum(diff * diff) * (1.0 / _K)
        svec = jnp.where((lane == i) & (sub == 0), shift, svec)
        traj_ref[s, i] = new
        return new, svec

    def body(i, carry):
        out = []
        for s in range(_PER):
            cent, svec = carry[2 * s], carry[2 * s + 1]
            new, svec = step(i, s, cent, svec, states[s][0])
            out += [new, svec]
        return tuple(out)

    init = []
    for s in range(_PER):
        init += [states[s][1], jnp.zeros((8, 128), jnp.float32)]
    fin = jax.lax.fori_loop(0, _ITERS, body, tuple(init))
    for s in range(_PER):
        shift_ref[s] = fin[2 * s + 1]


def _init_indices(b, n):
    def build():
        keys = jax.random.split(jax.random.key(42), b)
        idx = jnp.stack([jax.random.permutation(k, n)[:_K] for k in keys])
        return idx.astype(jnp.int32)
    try:
        with jax.ensure_compile_time_eval():
            return build()
    except Exception:
        return build()


@functools.partial(jax.jit, static_argnames=())
def kernel(x):
    b, h, w, c = x.shape
    n = h * w
    pts = x.reshape(b, n, c)
    idx = _init_indices(b, n)

    traj, shifts = pl.pallas_call(
        _kmeans_body,
        grid_spec=pltpu.PrefetchScalarGridSpec(
            num_scalar_prefetch=1,
            grid=(b // _PER,),
            in_specs=[
                pl.BlockSpec((_PER, n, c), lambda i, idx_ref: (i, 0, 0)),
            ],
            out_specs=[
                pl.BlockSpec((_PER, _ITERS, _K, c),
                             lambda i, idx_ref: (i, 0, 0, 0)),
                pl.BlockSpec((_PER, 8, 128), lambda i, idx_ref: (i, 0, 0)),
            ],
            scratch_shapes=[
                pltpu.VMEM((_PER, c, n), jnp.bfloat16),
                pltpu.VMEM((_PER, n, c), jnp.bfloat16),
            ],
        ),
        out_shape=[
            jax.ShapeDtypeStruct((b, _ITERS, _K, c), jnp.float32),
            jax.ShapeDtypeStruct((b, 8, 128), jnp.float32),
        ],
        compiler_params=pltpu.CompilerParams(
            dimension_semantics=("parallel",)),
    )(idx, pts)

    s = jnp.mean(shifts[:, 0, :_ITERS], axis=0)
    le = s <= _TOL
    t = jnp.where(jnp.any(le), jnp.argmax(le), _ITERS - 1)
    return jax.lax.dynamic_index_in_dim(traj, t, axis=1, keepdims=False)
